# baseline (speedup 1.0000x reference)
import functools

import jax
import jax.numpy as jnp
from jax import lax
from jax.experimental import pallas as pl
from jax.experimental.pallas import tpu as pltpu
from jax.experimental.pallas import tpu_sc as plsc

L = 16
C = 16
D = 64
VOCAB = 1000
NPOS = 16
NW = 32
ROWS = 1024 * 50
ROWS_PER_TILE = ROWS // NW
CHUNK = 320
NCHUNK = ROWS_PER_TILE // CHUNK
BLOCKS = CHUNK // L


def _sc_body(tok_hbm, pos_hbm, wword_hbm, wpos_hbm, out_hbm,
             wword_v, wpos_v, tok_v, pos_v, out_v):
    wid = lax.axis_index("s") * 2 + lax.axis_index("c")
    base = wid * ROWS_PER_TILE

    pltpu.sync_copy(wword_hbm, wword_v)
    pltpu.sync_copy(wpos_hbm, wpos_v)

    riota = lax.broadcasted_iota(jnp.int32, (L,), 0)

    for chunk in range(NCHUNK):
        r0 = base + chunk * CHUNK
        pltpu.sync_copy(tok_hbm.at[pl.ds(r0 * C, CHUNK * C)], tok_v)
        pltpu.sync_copy(pos_hbm.at[pl.ds(r0 * C, CHUNK * C)], pos_v)

        def block_body(b, carry):
            rb = b * L + riota
            rb_c = rb * C
            rb_d = rb * D
            tok_c = [plsc.load_gather(tok_v, [rb_c + c]) * D for c in range(C)]
            pos_c = [plsc.load_gather(pos_v, [rb_c + c]) * D for c in range(C)]

            @plsc.parallel_loop(0, D, unroll=8)
            def jbody(j):
                vals = [plsc.load_gather(wword_v, [tok_c[c] + j])
                        for c in range(C)]
                vals += [plsc.load_gather(wpos_v, [pos_c[c] + j])
                         for c in range(C)]
                while len(vals) > 1:
                    vals = [vals[i] + vals[i + 1] for i in range(0, len(vals) - 1, 2)] \
                        + ([vals[-1]] if len(vals) % 2 else [])
                plsc.store_scatter(out_v, [rb_d + j], vals[0])

            return carry

        lax.fori_loop(0, BLOCKS, block_body, 0)
        pltpu.sync_copy(out_v, out_hbm.at[pl.ds(r0 * D, CHUNK * D)])


@functools.partial(jax.jit, static_argnames=())
def kernel(token_ids, position_ids, W_word, W_pos):
    B, W, _ = token_ids.shape
    tok = token_ids.reshape(ROWS * C).astype(jnp.int32)
    pos = position_ids.reshape(ROWS * C).astype(jnp.int32)

    mesh = plsc.VectorSubcoreMesh(core_axis_name="c", subcore_axis_name="s")
    out = pl.kernel(
        _sc_body,
        out_type=jax.ShapeDtypeStruct((ROWS * D,), jnp.float32),
        mesh=mesh,
        compiler_params=pltpu.CompilerParams(needs_layout_passes=False),
        scratch_types=[
            pltpu.VMEM((VOCAB * D,), jnp.float32),
            pltpu.VMEM((NPOS * D,), jnp.float32),
            pltpu.VMEM((CHUNK * C,), jnp.int32),
            pltpu.VMEM((CHUNK * C,), jnp.int32),
            pltpu.VMEM((CHUNK * D,), jnp.float32),
        ],
    )(tok, pos, W_word.reshape(VOCAB * D), W_pos.reshape(NPOS * D))
    return out.reshape(B, W, D)


# speedup vs baseline: 5.4330x; 5.4330x over previous
import functools

import jax
import jax.numpy as jnp
from jax import lax
from jax.experimental import pallas as pl
from jax.experimental.pallas import tpu as pltpu
from jax.experimental.pallas import tpu_sc as plsc

L = 16
C = 16
D = 64
VOCAB = 1000
NPOS = 16
NW = 32
ROWS = 1024 * 50
ROWS_PER_TILE = ROWS // NW
CHUNK = 320
NCHUNK = ROWS_PER_TILE // CHUNK
BLOCKS = CHUNK // L

_GATHER_DNUMS = jax.lax.GatherDimensionNumbers(
    offset_dims=(), collapsed_slice_dims=(0,), start_index_map=(0,))


def _lane_bcast(vec, idx):
    return lax.gather(vec, idx[:, None], _GATHER_DNUMS, (1,),
                      mode=lax.GatherScatterMode.PROMISE_IN_BOUNDS)


def _sc_body(tok_hbm, pos_hbm, wword_hbm, wpos_hbm, out_hbm,
             wword_v, wpos_v, tok_v, pos_v, out_v):
    wid = lax.axis_index("s") * 2 + lax.axis_index("c")
    base = wid * ROWS_PER_TILE

    pltpu.sync_copy(wword_hbm, wword_v)
    pltpu.sync_copy(wpos_hbm, wpos_v)

    riota = lax.broadcasted_iota(jnp.int32, (L,), 0)
    colk = [riota + k * L for k in range(D // L)]

    for chunk in range(NCHUNK):
        r0 = base + chunk * CHUNK
        pltpu.sync_copy(tok_hbm.at[pl.ds(r0 * C, CHUNK * C)], tok_v)
        pltpu.sync_copy(pos_hbm.at[pl.ds(r0 * C, CHUNK * C)], pos_v)

        @plsc.parallel_loop(0, CHUNK, unroll=2)
        def row_body(r):
            tok_row = tok_v[pl.ds(r * C, C)]
            pos_row = pos_v[pl.ds(r * C, C)]
            accs = [jnp.zeros((L,), jnp.float32) for _ in range(D // L)]
            for c in range(C):
                cv = jnp.full((L,), c, jnp.int32)
                t = _lane_bcast(tok_row, cv) * D
                p = _lane_bcast(pos_row, cv) * D
                for k in range(D // L):
                    accs[k] = accs[k] + plsc.load_gather(wword_v, [t + colk[k]])
                    accs[k] = accs[k] + plsc.load_gather(wpos_v, [p + colk[k]])
            for k in range(D // L):
                out_v[pl.ds(r * D + k * L, L)] = accs[k]

        pltpu.sync_copy(out_v, out_hbm.at[pl.ds(r0 * D, CHUNK * D)])


@functools.partial(jax.jit, static_argnames=())
def kernel(token_ids, position_ids, W_word, W_pos):
    B, W, _ = token_ids.shape
    tok = token_ids.reshape(ROWS * C).astype(jnp.int32)
    pos = position_ids.reshape(ROWS * C).astype(jnp.int32)

    mesh = plsc.VectorSubcoreMesh(core_axis_name="c", subcore_axis_name="s")
    out = pl.kernel(
        _sc_body,
        out_type=jax.ShapeDtypeStruct((ROWS * D,), jnp.float32),
        mesh=mesh,
        compiler_params=pltpu.CompilerParams(needs_layout_passes=False),
        scratch_types=[
            pltpu.VMEM((VOCAB * D,), jnp.float32),
            pltpu.VMEM((NPOS * D,), jnp.float32),
            pltpu.VMEM((CHUNK * C,), jnp.int32),
            pltpu.VMEM((CHUNK * C,), jnp.int32),
            pltpu.VMEM((CHUNK * D,), jnp.float32),
        ],
    )(tok, pos, W_word.reshape(VOCAB * D), W_pos.reshape(NPOS * D))
    return out.reshape(B, W, D)


# speedup vs baseline: 5.5195x; 1.0159x over previous
import functools

import jax
import jax.numpy as jnp
from jax import lax
from jax.experimental import pallas as pl
from jax.experimental.pallas import tpu as pltpu
from jax.experimental.pallas import tpu_sc as plsc

L = 16
C = 16
D = 64
VOCAB = 1000
NPOS = 16
NW = 32
ROWS = 1024 * 50
ROWS_PER_TILE = ROWS // NW
CHUNK = 320
NCHUNK = ROWS_PER_TILE // CHUNK
BLOCKS = CHUNK // L

_GATHER_DNUMS = jax.lax.GatherDimensionNumbers(
    offset_dims=(), collapsed_slice_dims=(0,), start_index_map=(0,))


def _lane_bcast(vec, idx):
    return lax.gather(vec, idx[:, None], _GATHER_DNUMS, (1,),
                      mode=lax.GatherScatterMode.PROMISE_IN_BOUNDS)


def _sc_body(tok_hbm, pos_hbm, wword_hbm, wpos_hbm, out_hbm,
             wword_v, wpos_v, tok_v, pos_v, out_v):
    wid = lax.axis_index("s") * 2 + lax.axis_index("c")
    base = wid * ROWS_PER_TILE

    pltpu.sync_copy(wword_hbm, wword_v)
    pltpu.sync_copy(wpos_hbm, wpos_v)

    riota = lax.broadcasted_iota(jnp.int32, (L,), 0)
    colk = [riota + k * L for k in range(D // L)]

    for chunk in range(NCHUNK):
        r0 = base + chunk * CHUNK
        pltpu.sync_copy(tok_hbm.at[pl.ds(r0 * C, CHUNK * C)], tok_v)
        pltpu.sync_copy(pos_hbm.at[pl.ds(r0 * C, CHUNK * C)], pos_v)

        @plsc.parallel_loop(0, CHUNK, unroll=4)
        def row_body(r):
            tok_row = tok_v[pl.ds(r * C, C)]
            pos_row = pos_v[pl.ds(r * C, C)]
            accs = [jnp.zeros((L,), jnp.float32) for _ in range(D // L)]
            for c in range(C):
                t = tok_row[c] * D
                p = pos_row[c] * D
                for k in range(D // L):
                    accs[k] = accs[k] + wword_v[pl.ds(t + k * L, L)]
                    accs[k] = accs[k] + wpos_v[pl.ds(p + k * L, L)]
            for k in range(D // L):
                out_v[pl.ds(r * D + k * L, L)] = accs[k]

        pltpu.sync_copy(out_v, out_hbm.at[pl.ds(r0 * D, CHUNK * D)])


@functools.partial(jax.jit, static_argnames=())
def kernel(token_ids, position_ids, W_word, W_pos):
    B, W, _ = token_ids.shape
    tok = token_ids.reshape(ROWS * C).astype(jnp.int32)
    pos = position_ids.reshape(ROWS * C).astype(jnp.int32)

    mesh = plsc.VectorSubcoreMesh(core_axis_name="c", subcore_axis_name="s")
    out = pl.kernel(
        _sc_body,
        out_type=jax.ShapeDtypeStruct((ROWS * D,), jnp.float32),
        mesh=mesh,
        compiler_params=pltpu.CompilerParams(needs_layout_passes=False),
        scratch_types=[
            pltpu.VMEM((VOCAB * D,), jnp.float32),
            pltpu.VMEM((NPOS * D,), jnp.float32),
            pltpu.VMEM((CHUNK * C,), jnp.int32),
            pltpu.VMEM((CHUNK * C,), jnp.int32),
            pltpu.VMEM((CHUNK * D,), jnp.float32),
        ],
    )(tok, pos, W_word.reshape(VOCAB * D), W_pos.reshape(NPOS * D))
    return out.reshape(B, W, D)


# speedup vs baseline: 5.7446x; 1.0408x over previous
import functools

import jax
import jax.numpy as jnp
from jax import lax
from jax.experimental import pallas as pl
from jax.experimental.pallas import tpu as pltpu
from jax.experimental.pallas import tpu_sc as plsc

L = 16
C = 16
D = 64
VOCAB = 1000
NPOS = 16
NW = 32
ROWS = 1024 * 50
ROWS_PER_TILE = ROWS // NW
CHUNK = 320
NCHUNK = ROWS_PER_TILE // CHUNK
BLOCKS = CHUNK // L

_GATHER_DNUMS = jax.lax.GatherDimensionNumbers(
    offset_dims=(), collapsed_slice_dims=(0,), start_index_map=(0,))


def _lane_bcast(vec, idx):
    return lax.gather(vec, idx[:, None], _GATHER_DNUMS, (1,),
                      mode=lax.GatherScatterMode.PROMISE_IN_BOUNDS)


def _sc_body(tok_hbm, pos_hbm, wword_hbm, wpos_hbm, out_hbm,
             wword_v, wpos_v, tok_v, pos_v, out_v):
    wid = lax.axis_index("s") * 2 + lax.axis_index("c")
    base = wid * ROWS_PER_TILE

    pltpu.sync_copy(wword_hbm, wword_v)
    pltpu.sync_copy(wpos_hbm, wpos_v)

    riota = lax.broadcasted_iota(jnp.int32, (L,), 0)
    colk = [riota + k * L for k in range(D // L)]

    for chunk in range(NCHUNK):
        r0 = base + chunk * CHUNK
        pltpu.sync_copy(tok_hbm.at[pl.ds(r0 * C, CHUNK * C)], tok_v)
        pltpu.sync_copy(pos_hbm.at[pl.ds(r0 * C, CHUNK * C)], pos_v)

        @plsc.parallel_loop(0, CHUNK, unroll=4)
        def row_body(r):
            tok_row = tok_v[pl.ds(r * C, C)]
            pos_row = pos_v[pl.ds(r * C, C)]
            accs = [[None] * 4 for _ in range(D // L)]
            for c in range(C):
                t = tok_row[c] * D
                p = pos_row[c] * D
                for k in range(D // L):
                    w = wword_v[pl.ds(t + k * L, L)]
                    q = wpos_v[pl.ds(p + k * L, L)]
                    s0, s1 = (c % 2), 2 + (c % 2)
                    accs[k][s0] = w if accs[k][s0] is None else accs[k][s0] + w
                    accs[k][s1] = q if accs[k][s1] is None else accs[k][s1] + q
            for k in range(D // L):
                a = accs[k]
                out_v[pl.ds(r * D + k * L, L)] = (a[0] + a[1]) + (a[2] + a[3])

        pltpu.sync_copy(out_v, out_hbm.at[pl.ds(r0 * D, CHUNK * D)])


@functools.partial(jax.jit, static_argnames=())
def kernel(token_ids, position_ids, W_word, W_pos):
    B, W, _ = token_ids.shape
    tok = token_ids.reshape(ROWS * C).astype(jnp.int32)
    pos = position_ids.reshape(ROWS * C).astype(jnp.int32)

    mesh = plsc.VectorSubcoreMesh(core_axis_name="c", subcore_axis_name="s")
    out = pl.kernel(
        _sc_body,
        out_type=jax.ShapeDtypeStruct((ROWS * D,), jnp.float32),
        mesh=mesh,
        compiler_params=pltpu.CompilerParams(needs_layout_passes=False),
        scratch_types=[
            pltpu.VMEM((VOCAB * D,), jnp.float32),
            pltpu.VMEM((NPOS * D,), jnp.float32),
            pltpu.VMEM((CHUNK * C,), jnp.int32),
            pltpu.VMEM((CHUNK * C,), jnp.int32),
            pltpu.VMEM((CHUNK * D,), jnp.float32),
        ],
    )(tok, pos, W_word.reshape(VOCAB * D), W_pos.reshape(NPOS * D))
    return out.reshape(B, W, D)


# speedup vs baseline: 9.4564x; 1.6462x over previous
import functools

import jax
import jax.numpy as jnp
from jax import lax
from jax.experimental import pallas as pl
from jax.experimental.pallas import tpu as pltpu
from jax.experimental.pallas import tpu_sc as plsc

L = 16
C = 16
D = 64
PW = D // 2
G = D // 32
VOCAB = 1000
NPOS = 16
NW = 32
ROWS = 1024 * 50
ROWS_PER_TILE = ROWS // NW
CHUNK = 320
NCHUNK = ROWS_PER_TILE // CHUNK

_HI = jnp.int32(-65536)


def _pack_table(w):
    v = w.shape[0]
    bits = lax.bitcast_convert_type(w.astype(jnp.bfloat16), jnp.uint16)
    bits = bits.reshape(v, G, 2, L).astype(jnp.uint32)
    words = bits[:, :, 0, :] | (bits[:, :, 1, :] << 16)
    return lax.bitcast_convert_type(words, jnp.int32).reshape(v * PW)


def _sc_body(tok_hbm, pos_hbm, wword_hbm, wpos_hbm, out_hbm,
             wword_v, wpos_v, tok_v, pos_v, out_v):
    wid = lax.axis_index("s") * 2 + lax.axis_index("c")
    base = wid * ROWS_PER_TILE

    pltpu.sync_copy(wword_hbm, wword_v)
    pltpu.sync_copy(wpos_hbm, wpos_v)

    for chunk in range(NCHUNK):
        r0 = base + chunk * CHUNK
        pltpu.sync_copy(tok_hbm.at[pl.ds(r0 * C, CHUNK * C)], tok_v)
        pltpu.sync_copy(pos_hbm.at[pl.ds(r0 * C, CHUNK * C)], pos_v)

        @plsc.parallel_loop(0, CHUNK, unroll=4)
        def row_body(r):
            tok_row = tok_v[pl.ds(r * C, C)]
            pos_row = pos_v[pl.ds(r * C, C)]
            accw = [[None, None] for _ in range(G)]
            accp = [[None, None] for _ in range(G)]
            for c in range(C):
                t = tok_row[c] * PW
                p = pos_row[c] * PW
                for g in range(G):
                    w = wword_v[pl.ds(t + g * L, L)]
                    q = wpos_v[pl.ds(p + g * L, L)]
                    for h, (wv, qv) in enumerate((
                            (plsc.bitcast(w << 16, jnp.float32),
                             plsc.bitcast(q << 16, jnp.float32)),
                            (plsc.bitcast(w & _HI, jnp.float32),
                             plsc.bitcast(q & _HI, jnp.float32)))):
                        accw[g][h] = wv if accw[g][h] is None else accw[g][h] + wv
                        accp[g][h] = qv if accp[g][h] is None else accp[g][h] + qv
            for g in range(G):
                for h in range(2):
                    out_v[pl.ds(r * D + g * 32 + h * L, L)] = \
                        accw[g][h] + accp[g][h]

        pltpu.sync_copy(out_v, out_hbm.at[pl.ds(r0 * D, CHUNK * D)])


@functools.partial(jax.jit, static_argnames=())
def kernel(token_ids, position_ids, W_word, W_pos):
    B, W, _ = token_ids.shape
    tok = token_ids.reshape(ROWS * C).astype(jnp.int32)
    pos = position_ids.reshape(ROWS * C).astype(jnp.int32)
    wword = _pack_table(W_word)
    wpos = _pack_table(W_pos)

    mesh = plsc.VectorSubcoreMesh(core_axis_name="c", subcore_axis_name="s")
    out = pl.kernel(
        _sc_body,
        out_type=jax.ShapeDtypeStruct((ROWS * D,), jnp.float32),
        mesh=mesh,
        compiler_params=pltpu.CompilerParams(needs_layout_passes=False),
        scratch_types=[
            pltpu.VMEM((VOCAB * PW,), jnp.int32),
            pltpu.VMEM((NPOS * PW,), jnp.int32),
            pltpu.VMEM((CHUNK * C,), jnp.int32),
            pltpu.VMEM((CHUNK * C,), jnp.int32),
            pltpu.VMEM((CHUNK * D,), jnp.float32),
        ],
    )(tok, pos, wword, wpos)
    return out.reshape(B, W, D)


# speedup vs baseline: 9.5460x; 1.0095x over previous
import functools

import jax
import jax.numpy as jnp
from jax import lax
from jax.experimental import pallas as pl
from jax.experimental.pallas import tpu as pltpu
from jax.experimental.pallas import tpu_sc as plsc

L = 16
C = 16
D = 64
PW = D // 2
G = D // 32
VOCAB = 1000
NPOS = 16
NW = 32
ROWS = 1024 * 50
ROWS_PER_TILE = ROWS // NW
CHUNK = 320
NCHUNK = ROWS_PER_TILE // CHUNK

_HI = jnp.int32(-65536)


def _pack_table(w):
    v = w.shape[0]
    bits = lax.bitcast_convert_type(w.astype(jnp.bfloat16), jnp.uint16)
    bits = bits.reshape(v, G, 2, L).astype(jnp.uint32)
    words = bits[:, :, 0, :] | (bits[:, :, 1, :] << 16)
    return lax.bitcast_convert_type(words, jnp.int32).reshape(v * PW)


def _sc_body(tok_hbm, pos_hbm, wword_hbm, wpos_hbm, out_hbm,
             wword_v, wpos_v, tok_v, pos_v, out_v):
    wid = lax.axis_index("s") * 2 + lax.axis_index("c")
    base = wid * ROWS_PER_TILE

    pltpu.sync_copy(wword_hbm, wword_v)
    pltpu.sync_copy(wpos_hbm, wpos_v)

    for chunk in range(NCHUNK):
        r0 = base + chunk * CHUNK
        pltpu.sync_copy(tok_hbm.at[pl.ds(r0 * C, CHUNK * C)], tok_v)
        pltpu.sync_copy(pos_hbm.at[pl.ds(r0 * C, CHUNK * C)], pos_v)

        @plsc.parallel_loop(0, CHUNK, unroll=8)
        def row_body(r):
            tok_row = tok_v[pl.ds(r * C, C)]
            pos_row = pos_v[pl.ds(r * C, C)]
            accw = [[None, None] for _ in range(G)]
            accp = [[None, None] for _ in range(G)]
            for c in range(C):
                t = tok_row[c] * PW
                p = pos_row[c] * PW
                for g in range(G):
                    w = wword_v[pl.ds(t + g * L, L)]
                    q = wpos_v[pl.ds(p + g * L, L)]
                    for h, (wv, qv) in enumerate((
                            (plsc.bitcast(w << 16, jnp.float32),
                             plsc.bitcast(q << 16, jnp.float32)),
                            (plsc.bitcast(w & _HI, jnp.float32),
                             plsc.bitcast(q & _HI, jnp.float32)))):
                        accw[g][h] = wv if accw[g][h] is None else accw[g][h] + wv
                        accp[g][h] = qv if accp[g][h] is None else accp[g][h] + qv
            for g in range(G):
                for h in range(2):
                    out_v[pl.ds(r * D + g * 32 + h * L, L)] = \
                        accw[g][h] + accp[g][h]

        pltpu.sync_copy(out_v, out_hbm.at[pl.ds(r0 * D, CHUNK * D)])


@functools.partial(jax.jit, static_argnames=())
def kernel(token_ids, position_ids, W_word, W_pos):
    B, W, _ = token_ids.shape
    tok = token_ids.reshape(ROWS * C).astype(jnp.int32)
    pos = position_ids.reshape(ROWS * C).astype(jnp.int32)
    wword = _pack_table(W_word)
    wpos = _pack_table(W_pos)

    mesh = plsc.VectorSubcoreMesh(core_axis_name="c", subcore_axis_name="s")
    out = pl.kernel(
        _sc_body,
        out_type=jax.ShapeDtypeStruct((ROWS * D,), jnp.float32),
        mesh=mesh,
        compiler_params=pltpu.CompilerParams(needs_layout_passes=False),
        scratch_types=[
            pltpu.VMEM((VOCAB * PW,), jnp.int32),
            pltpu.VMEM((NPOS * PW,), jnp.int32),
            pltpu.VMEM((CHUNK * C,), jnp.int32),
            pltpu.VMEM((CHUNK * C,), jnp.int32),
            pltpu.VMEM((CHUNK * D,), jnp.float32),
        ],
    )(tok, pos, wword, wpos)
    return out.reshape(B, W, D)


# speedup vs baseline: 10.3355x; 1.0827x over previous
import functools

import jax
import jax.numpy as jnp
from jax import lax
from jax.experimental import pallas as pl
from jax.experimental.pallas import tpu as pltpu
from jax.experimental.pallas import tpu_sc as plsc

L = 16
C = 16
D = 64
PW = D // 2
G = D // 32
VOCAB = 1000
NPOS = 16
NW = 32
ROWS = 1024 * 50
ROWS_PER_TILE = ROWS // NW
CHUNK = 320
NCHUNK = ROWS_PER_TILE // CHUNK

_HI = jnp.int32(-65536)

RPV = 128 // C
PROWS = ROWS // RPV
PCOLS = RPV * D
TBR = 128
TNB = PROWS // TBR


def _pack_word_table(w):
    v = w.shape[0]
    bits = lax.bitcast_convert_type(w.astype(jnp.bfloat16), jnp.uint16)
    bits = bits.reshape(v, G, 2, L).astype(jnp.uint32)
    words = bits[:, :, 0, :] | (bits[:, :, 1, :] << 16)
    return lax.bitcast_convert_type(words, jnp.int32).reshape(v * PW)


def _sc_body(tok_hbm, wword_hbm, out_hbm, wword_v, tok_v, out_v):
    wid = lax.axis_index("s") * 2 + lax.axis_index("c")
    base = wid * ROWS_PER_TILE

    pltpu.sync_copy(wword_hbm, wword_v)

    for chunk in range(NCHUNK):
        r0 = base + chunk * CHUNK
        pltpu.sync_copy(tok_hbm.at[pl.ds(r0 * C, CHUNK * C)], tok_v)

        @plsc.parallel_loop(0, CHUNK, unroll=8)
        def row_body(r):
            tok_row = tok_v[pl.ds(r * C, C)]
            acc = [[[None, None], [None, None]] for _ in range(G)]
            for c in range(C):
                t = tok_row[c] * PW
                s = c % 2
                for g in range(G):
                    w = wword_v[pl.ds(t + g * L, L)]
                    for h, wv in enumerate((
                            plsc.bitcast(w << 16, jnp.float32),
                            plsc.bitcast(w & _HI, jnp.float32))):
                        a = acc[g][h]
                        a[s] = wv if a[s] is None else a[s] + wv
            for g in range(G):
                for h in range(2):
                    out_v[pl.ds(r * D + g * 32 + h * L, L)] = \
                        acc[g][h][0] + acc[g][h][1]

        pltpu.sync_copy(out_v, out_hbm.at[pl.ds(r0 * D, CHUNK * D)])


def _tc_body(prow_ref, ball_ref, out_ref):
    pos_blk = prow_ref[...]
    acc = jnp.zeros((TBR, PCOLS), jnp.float32)
    for i in range(NPOS):
        oh = (pos_blk == i).astype(jnp.float32)
        acc = acc + lax.dot_general(
            oh, ball_ref[i],
            (((1,), (0,)), ((), ())),
            preferred_element_type=jnp.float32)
    out_ref[...] = acc


@functools.partial(jax.jit, static_argnames=())
def kernel(token_ids, position_ids, W_word, W_pos):
    B, W, _ = token_ids.shape
    tok = token_ids.reshape(ROWS * C).astype(jnp.int32)
    prow = position_ids.reshape(PROWS, 128).astype(jnp.int32)
    wword = _pack_word_table(W_word)

    sel = (jnp.arange(128)[:, None] // C == jnp.arange(PCOLS)[None, :] // D)
    ball = sel.astype(jnp.float32)[None] * jnp.tile(W_pos, (1, RPV))[:, None, :]

    mesh = plsc.VectorSubcoreMesh(core_axis_name="c", subcore_axis_name="s")
    word_part = pl.kernel(
        _sc_body,
        out_type=jax.ShapeDtypeStruct((ROWS * D,), jnp.float32),
        mesh=mesh,
        compiler_params=pltpu.CompilerParams(needs_layout_passes=False),
        scratch_types=[
            pltpu.VMEM((VOCAB * PW,), jnp.int32),
            pltpu.VMEM((CHUNK * C,), jnp.int32),
            pltpu.VMEM((CHUNK * D,), jnp.float32),
        ],
    )(tok, wword)

    pos_part = pl.pallas_call(
        _tc_body,
        grid=(TNB,),
        in_specs=[
            pl.BlockSpec((TBR, 128), lambda b: (b, 0)),
            pl.BlockSpec((NPOS, 128, PCOLS), lambda b: (0, 0, 0)),
        ],
        out_specs=pl.BlockSpec((TBR, PCOLS), lambda b: (b, 0)),
        out_shape=jax.ShapeDtypeStruct((PROWS, PCOLS), jnp.float32),
    )(prow, ball)

    out = word_part.reshape(B, W, D) + pos_part.reshape(B, W, D)
    return out
